# baseline (speedup 1.0000x reference)
import functools

import jax
import jax.numpy as jnp
from jax import lax
from jax.experimental import pallas as pl
from jax.experimental.pallas import tpu as pltpu
from jax.experimental.pallas import tpu_sc as plsc

H_DIM = 64
BATCH = 16384
_NC = 2
_NS = 16
_NW = _NC * _NS
_B_PER_W = BATCH // _NW
_CHUNK = 128
_N_CHUNKS = _B_PER_W // _CHUNK


def _emb_body(idx_hbm, table_hbm, out_hbm, idx_v, out_v, sem):
    wid = lax.axis_index("s") * _NC + lax.axis_index("c")
    base = wid * _B_PER_W
    pltpu.sync_copy(idx_hbm.at[pl.ds(base, _B_PER_W)], idx_v)

    copies = [
        pltpu.async_copy(
            table_hbm.at[idx_v.at[pl.ds(c * _CHUNK, _CHUNK)]],
            out_v.at[pl.ds(c * _CHUNK, _CHUNK)],
            sem,
        )
        for c in range(_N_CHUNKS)
    ]
    for cp in copies:
        cp.wait()

    pltpu.sync_copy(out_v, out_hbm.at[pl.ds(base, _B_PER_W)])


@jax.jit
def kernel(node_id, table):
    idx = jnp.asarray(node_id, jnp.int32)
    f = functools.partial(
        pl.kernel,
        mesh=plsc.VectorSubcoreMesh(core_axis_name="c", subcore_axis_name="s"),
        out_type=jax.ShapeDtypeStruct((BATCH, H_DIM), jnp.float32),
        compiler_params=pltpu.CompilerParams(use_tc_tiling_on_sc=False),
        scratch_types=[
            pltpu.VMEM((_B_PER_W,), jnp.int32),
            pltpu.VMEM((_B_PER_W, H_DIM), jnp.float32),
            pltpu.SemaphoreType.DMA,
        ],
    )(_emb_body)
    return f(idx, table)


# speedup vs baseline: 2.6027x; 2.6027x over previous
import functools

import jax
import jax.numpy as jnp
from jax import lax
from jax.experimental import pallas as pl
from jax.experimental.pallas import tpu as pltpu
from jax.experimental.pallas import tpu_sc as plsc

H_DIM = 64
BATCH = 16384
_NC = 2
_NS = 16
_NW = _NC * _NS
_B_PER_W = BATCH // _NW
_G = 16
_N_G = _B_PER_W // _G
_HP = 32
_N_HP = H_DIM // _HP


def _emb_body(idx_hbm, table_hbm, out_hbm, idx_v, staged_v, out_v, sem):
    wid = lax.axis_index("s") * _NC + lax.axis_index("c")
    base = wid * _B_PER_W
    pltpu.sync_copy(idx_hbm.at[pl.ds(base, _B_PER_W)], idx_v)

    lanes = lax.iota(jnp.int32, 16)

    def _group(g, carry):
        r = idx_v[pl.ds(g * _G, _G)]
        a128 = lax.shift_left(lax.shift_right_logical(r, 7), 7)
        l128 = lax.bitwise_and(r, 127)
        starts = [pl.multiple_of(a128[k], 128) for k in range(_G)]

        for h in range(_N_HP):
            copies = [
                pltpu.async_copy(
                    table_hbm.at[pl.ds(h * _HP, _HP), pl.ds(starts[k], 128)],
                    staged_v.at[k],
                    sem,
                )
                for k in range(_G)
            ]
            for cp in copies:
                cp.wait()

            for cc in range(_HP):
                vals = plsc.load_gather(
                    staged_v, [lanes, jnp.full((16,), cc, jnp.int32), l128]
                )
                out_v[h * _HP + cc, pl.ds(g * _G, _G)] = vals
        return carry

    lax.fori_loop(0, _N_G, _group, 0)
    pltpu.sync_copy(out_v, out_hbm.at[:, pl.ds(base, _B_PER_W)])


@jax.jit
def kernel(node_id, table):
    idx = jnp.asarray(node_id, jnp.int32)
    table_t = table.T
    f = functools.partial(
        pl.kernel,
        mesh=plsc.VectorSubcoreMesh(core_axis_name="c", subcore_axis_name="s"),
        out_type=jax.ShapeDtypeStruct((H_DIM, BATCH), jnp.float32),
        compiler_params=pltpu.CompilerParams(needs_layout_passes=False),
        scratch_types=[
            pltpu.VMEM((_B_PER_W,), jnp.int32),
            pltpu.VMEM((_G, _HP, 128), jnp.float32),
            pltpu.VMEM((H_DIM, _B_PER_W), jnp.float32),
            pltpu.SemaphoreType.DMA,
        ],
    )(_emb_body)
    out_t = f(idx, table_t)
    return out_t.T


# speedup vs baseline: 2.6176x; 1.0057x over previous
import functools

import jax
import jax.numpy as jnp
from jax import lax
from jax.experimental import pallas as pl
from jax.experimental.pallas import tpu as pltpu
from jax.experimental.pallas import tpu_sc as plsc

H_DIM = 64
BATCH = 16384
_NC = 2
_NS = 16
_NW = _NC * _NS
_B_PER_W = BATCH // _NW
_G = 16
_N_G = _B_PER_W // _G
_HP = 16
_N_HP = H_DIM // _HP


def _emb_body(idx_hbm, table_hbm, out_hbm, idx_v, staged_v, out_v, sem0, sem1):
    wid = lax.axis_index("s") * _NC + lax.axis_index("c")
    base = wid * _B_PER_W
    pltpu.sync_copy(idx_hbm.at[pl.ds(base, _B_PER_W)], idx_v)

    lanes = lax.iota(jnp.int32, 16)
    sems = (sem0, sem1)

    def starts_of(g):
        r = idx_v[pl.ds(g * _G, _G)]
        a128 = lax.shift_left(lax.shift_right_logical(r, 7), 7)
        return [pl.multiple_of(a128[k], 128) for k in range(_G)]

    def issue(starts, h, buf):
        for k in range(_G):
            pltpu.async_copy(
                table_hbm.at[pl.ds(h * _HP, _HP), pl.ds(starts[k], 128)],
                staged_v.at[buf, k],
                sems[buf],
            )

    def drain(buf):
        for _ in range(_G):
            pltpu.make_async_copy(
                table_hbm.at[pl.ds(0, _HP), pl.ds(0, 128)],
                staged_v.at[buf, 0],
                sems[buf],
            ).wait()

    def extract(l128, g, h, buf):
        for cc in range(_HP):
            vals = plsc.load_gather(
                staged_v.at[buf], [lanes, jnp.full((16,), cc, jnp.int32), l128]
            )
            out_v[h * _HP + cc, pl.ds(g * _G, _G)] = vals

    issue(starts_of(0), 0, 0)

    def _group(g, carry):
        r = idx_v[pl.ds(g * _G, _G)]
        l128 = lax.bitwise_and(r, 127)
        starts = starts_of(g)
        issue(starts, 1, 1)
        drain(0)
        extract(l128, g, 0, 0)
        issue(starts, 2, 0)
        drain(1)
        extract(l128, g, 1, 1)
        issue(starts, 3, 1)
        drain(0)
        extract(l128, g, 2, 0)
        g_next = lax.min(g + 1, _N_G - 1)
        issue(starts_of(g_next), 0, 0)
        drain(1)
        extract(l128, g, 3, 1)
        return carry

    lax.fori_loop(0, _N_G, _group, 0)
    drain(0)
    pltpu.sync_copy(out_v, out_hbm.at[:, pl.ds(base, _B_PER_W)])


@jax.jit
def kernel(node_id, table):
    idx = jnp.asarray(node_id, jnp.int32)
    table_t = table.T
    f = functools.partial(
        pl.kernel,
        mesh=plsc.VectorSubcoreMesh(core_axis_name="c", subcore_axis_name="s"),
        out_type=jax.ShapeDtypeStruct((H_DIM, BATCH), jnp.float32),
        compiler_params=pltpu.CompilerParams(needs_layout_passes=False),
        scratch_types=[
            pltpu.VMEM((_B_PER_W,), jnp.int32),
            pltpu.VMEM((2, _G, _HP, 128), jnp.float32),
            pltpu.VMEM((H_DIM, _B_PER_W), jnp.float32),
            pltpu.SemaphoreType.DMA,
            pltpu.SemaphoreType.DMA,
        ],
    )(_emb_body)
    out_t = f(idx, table_t)
    return out_t.T
